# baseline (speedup 1.0000x reference)
import functools

import jax
import jax.numpy as jnp
from jax import lax
from jax.experimental import pallas as pl
from jax.experimental.pallas import tpu as pltpu
from jax.experimental.pallas import tpu_sc as plsc

_V = 1_000_000
_D = 64
_B = 4096
_S = 200
_LANES = 16
_TW = 2048

_info = plsc.get_sparse_core_info()
_NC, _NS = _info.num_cores, _info.num_subcores
_NW = _NC * _NS
_BPW = _B // _NW

_CHUNK_STARTS = tuple(range(0, _S - _LANES, _LANES)) + (_S - _LANES,)


def _widen_body(embT_ref, out_ref):
    x = embT_ref[...]
    eye = jnp.eye(_D, dtype=jnp.float32)
    out_ref[:, 0:_D] = lax.dot_general(
        x, eye, (((0,), (0,)), ((), ())), preferred_element_type=jnp.float32
    )
    out_ref[:, _D:128] = jnp.zeros((_TW, _D), jnp.float32)


def _widen(embT):
    return pl.pallas_call(
        _widen_body,
        grid=(pl.cdiv(_V, _TW),),
        in_specs=[pl.BlockSpec((_D, _TW), lambda i: (0, i))],
        out_specs=pl.BlockSpec((_TW, 128), lambda i: (i, 0)),
        out_shape=jax.ShapeDtypeStruct((_V, 128), jnp.float32),
    )(embT)


def _body(idx_hbm, emb_hbm, out_hbm, tok_v, idxT_v, acc_v, sem):
    wid = lax.axis_index("s") * _NC + lax.axis_index("c")
    pltpu.sync_copy(idx_hbm.at[wid], tok_v)

    iota = lax.iota(jnp.int32, _LANES)
    jvecs = [(iota + c) * _BPW for c in _CHUNK_STARTS]

    def tr_row(b, carry):
        for c, jvec in zip(_CHUNK_STARTS, jvecs):
            x = tok_v[b, pl.ds(c, _LANES)] * 2
            plsc.store_scatter(idxT_v, [jvec + b], x)
        return carry

    lax.fori_loop(0, _BPW, tr_row, 0)

    zeros = jnp.zeros((_LANES,), jnp.float32)

    def zero_row(i, carry):
        for k in range(_D // _LANES):
            acc_v[i, pl.ds(k * _LANES, _LANES)] = zeros
        return carry

    lax.fori_loop(0, _BPW, zero_row, 0)

    def issue(j, carry):
        pltpu.async_copy(
            emb_hbm.at[idxT_v.at[pl.ds(j * _BPW, _BPW)]], acc_v, sem, add=True
        )
        return carry

    lax.fori_loop(0, _S, issue, 0)

    def drain(j, carry):
        pltpu.make_async_copy(
            emb_hbm.at[idxT_v.at[pl.ds(0, _BPW)]], acc_v, sem
        ).wait()
        return carry

    lax.fori_loop(0, _S, drain, 0)

    scale = jnp.full((_LANES,), 1.0 / _S, jnp.float32)

    def scale_row(i, carry):
        for k in range(_D // _LANES):
            sl = pl.ds(k * _LANES, _LANES)
            acc_v[i, sl] = acc_v[i, sl] * scale
        return carry

    lax.fori_loop(0, _BPW, scale_row, 0)

    pltpu.sync_copy(acc_v, out_hbm.at[pl.ds(wid * _BPW, _BPW)])


def kernel(text_tokens, emb):
    embp = _widen(emb.T).reshape(2 * _V, _D)
    idx3 = text_tokens.astype(jnp.int32).reshape(_NW, _BPW, _S)
    mesh = plsc.VectorSubcoreMesh(core_axis_name="c", subcore_axis_name="s")
    sc_call = pl.kernel(
        _body,
        out_type=jax.ShapeDtypeStruct((_B, _D), jnp.float32),
        mesh=mesh,
        scratch_types=[
            pltpu.VMEM((_BPW, _S), jnp.int32),
            pltpu.VMEM((_S * _BPW,), jnp.int32),
            pltpu.VMEM((_BPW, _D), jnp.float32),
            pltpu.SemaphoreType.DMA,
        ],
        compiler_params=pltpu.CompilerParams(
            use_tc_tiling_on_sc=False, needs_layout_passes=False
        ),
    )
    return sc_call(idx3, embp)


# speedup vs baseline: 1.5466x; 1.5466x over previous
import functools

import jax
import jax.numpy as jnp
from jax import lax
from jax.experimental import pallas as pl
from jax.experimental.pallas import tpu as pltpu
from jax.experimental.pallas import tpu_sc as plsc

_V = 1_000_000
_D = 64
_B = 4096
_S = 200
_LANES = 16
_TW = 8192

_info = plsc.get_sparse_core_info()
_NC, _NS = _info.num_cores, _info.num_subcores
_NW = _NC * _NS
_BPW = _B // _NW

_CHUNK_STARTS = tuple(range(0, _S - _LANES, _LANES)) + (_S - _LANES,)


def _widen_body(embT_ref, out_ref):
    x = embT_ref[...]
    eye = jnp.eye(_D, dtype=jnp.float32)
    out_ref[:, 0:_D] = lax.dot_general(
        x, eye, (((0,), (0,)), ((), ())), preferred_element_type=jnp.float32
    )
    out_ref[:, _D:128] = jnp.zeros((_TW, _D), jnp.float32)


def _widen(embT):
    return pl.pallas_call(
        _widen_body,
        grid=(pl.cdiv(_V, _TW),),
        in_specs=[pl.BlockSpec((_D, _TW), lambda i: (0, i))],
        out_specs=pl.BlockSpec((_TW, 128), lambda i: (i, 0)),
        out_shape=jax.ShapeDtypeStruct((_V, 128), jnp.float32),
    )(embT)


def _body(idx_hbm, emb_hbm, out_hbm, tok_v, idxT_v, acc_v, sem):
    wid = lax.axis_index("s") * _NC + lax.axis_index("c")
    pltpu.sync_copy(idx_hbm.at[wid], tok_v)

    iota = lax.iota(jnp.int32, _LANES)
    jvecs = [(iota + c) * _BPW for c in _CHUNK_STARTS]

    def tr_row(b, carry):
        for c, jvec in zip(_CHUNK_STARTS, jvecs):
            x = tok_v[b, pl.ds(c, _LANES)] * 2
            plsc.store_scatter(idxT_v, [jvec + b], x)
        return carry

    lax.fori_loop(0, _BPW, tr_row, 0)

    zeros = jnp.zeros((_LANES,), jnp.float32)

    def zero_row(i, carry):
        for k in range(_D // _LANES):
            acc_v[i, pl.ds(k * _LANES, _LANES)] = zeros
        return carry

    lax.fori_loop(0, _BPW, zero_row, 0)

    def issue(j, carry):
        pltpu.async_copy(
            emb_hbm.at[idxT_v.at[pl.ds(j * _BPW, _BPW)]], acc_v, sem, add=True
        )
        return carry

    lax.fori_loop(0, _S, issue, 0)

    def drain(j, carry):
        pltpu.make_async_copy(
            emb_hbm.at[idxT_v.at[pl.ds(0, _BPW)]], acc_v, sem
        ).wait()
        return carry

    lax.fori_loop(0, _S, drain, 0)

    scale = jnp.full((_LANES,), 1.0 / _S, jnp.float32)

    def scale_row(i, carry):
        for k in range(_D // _LANES):
            sl = pl.ds(k * _LANES, _LANES)
            acc_v[i, sl] = acc_v[i, sl] * scale
        return carry

    lax.fori_loop(0, _BPW, scale_row, 0)

    pltpu.sync_copy(acc_v, out_hbm.at[pl.ds(wid * _BPW, _BPW)])


def kernel(text_tokens, emb):
    embp = _widen(emb.T).reshape(2 * _V, _D)
    idx3 = text_tokens.astype(jnp.int32).reshape(_NW, _BPW, _S)
    mesh = plsc.VectorSubcoreMesh(core_axis_name="c", subcore_axis_name="s")
    sc_call = pl.kernel(
        _body,
        out_type=jax.ShapeDtypeStruct((_B, _D), jnp.float32),
        mesh=mesh,
        scratch_types=[
            pltpu.VMEM((_BPW, _S), jnp.int32),
            pltpu.VMEM((_S * _BPW,), jnp.int32),
            pltpu.VMEM((_BPW, _D), jnp.float32),
            pltpu.SemaphoreType.DMA,
        ],
        compiler_params=pltpu.CompilerParams(
            use_tc_tiling_on_sc=False, needs_layout_passes=False
        ),
    )
    return sc_call(idx3, embp)


# speedup vs baseline: 1.5699x; 1.0151x over previous
import functools

import jax
import jax.numpy as jnp
from jax import lax
from jax.experimental import pallas as pl
from jax.experimental.pallas import tpu as pltpu
from jax.experimental.pallas import tpu_sc as plsc

_V = 1_000_000
_D = 64
_B = 4096
_S = 200
_LANES = 16
_TW = 8192

_info = plsc.get_sparse_core_info()
_NC, _NS = _info.num_cores, _info.num_subcores
_NW = _NC * _NS
_BPW = _B // _NW

_CHUNK_STARTS = tuple(range(0, _S - _LANES, _LANES)) + (_S - _LANES,)


def _widen_body(embT_ref, out_ref):
    x = embT_ref[...]
    out_ref[:, 0:_D] = x.T
    out_ref[:, _D:128] = jnp.zeros((_TW, _D), jnp.float32)


def _widen(embT):
    return pl.pallas_call(
        _widen_body,
        grid=(pl.cdiv(_V, _TW),),
        in_specs=[pl.BlockSpec((_D, _TW), lambda i: (0, i))],
        out_specs=pl.BlockSpec((_TW, 128), lambda i: (i, 0)),
        out_shape=jax.ShapeDtypeStruct((_V, 128), jnp.float32),
    )(embT)


def _body(idx_hbm, emb_hbm, out_hbm, tok_v, idxT_v, acc_v, sem):
    wid = lax.axis_index("s") * _NC + lax.axis_index("c")
    pltpu.sync_copy(idx_hbm.at[wid], tok_v)

    iota = lax.iota(jnp.int32, _LANES)
    jvecs = [(iota + c) * _BPW for c in _CHUNK_STARTS]

    def tr_row(b, carry):
        for c, jvec in zip(_CHUNK_STARTS, jvecs):
            x = tok_v[b, pl.ds(c, _LANES)] * 2
            plsc.store_scatter(idxT_v, [jvec + b], x)
        return carry

    lax.fori_loop(0, _BPW, tr_row, 0)

    zeros = jnp.zeros((_LANES,), jnp.float32)

    def zero_row(i, carry):
        for k in range(_D // _LANES):
            acc_v[i, pl.ds(k * _LANES, _LANES)] = zeros
        return carry

    lax.fori_loop(0, _BPW, zero_row, 0)

    def issue(j, carry):
        pltpu.async_copy(
            emb_hbm.at[idxT_v.at[pl.ds(j * _BPW, _BPW)]], acc_v, sem, add=True
        )
        return carry

    lax.fori_loop(0, _S, issue, 0)

    def drain(j, carry):
        pltpu.make_async_copy(
            emb_hbm.at[idxT_v.at[pl.ds(0, _BPW)]], acc_v, sem
        ).wait()
        return carry

    lax.fori_loop(0, _S, drain, 0)

    scale = jnp.full((_LANES,), 1.0 / _S, jnp.float32)

    def scale_row(i, carry):
        for k in range(_D // _LANES):
            sl = pl.ds(k * _LANES, _LANES)
            acc_v[i, sl] = acc_v[i, sl] * scale
        return carry

    lax.fori_loop(0, _BPW, scale_row, 0)

    pltpu.sync_copy(acc_v, out_hbm.at[pl.ds(wid * _BPW, _BPW)])


def kernel(text_tokens, emb):
    embp = _widen(emb.T).reshape(2 * _V, _D)
    idx3 = text_tokens.astype(jnp.int32).reshape(_NW, _BPW, _S)
    mesh = plsc.VectorSubcoreMesh(core_axis_name="c", subcore_axis_name="s")
    sc_call = pl.kernel(
        _body,
        out_type=jax.ShapeDtypeStruct((_B, _D), jnp.float32),
        mesh=mesh,
        scratch_types=[
            pltpu.VMEM((_BPW, _S), jnp.int32),
            pltpu.VMEM((_S * _BPW,), jnp.int32),
            pltpu.VMEM((_BPW, _D), jnp.float32),
            pltpu.SemaphoreType.DMA,
        ],
        compiler_params=pltpu.CompilerParams(
            use_tc_tiling_on_sc=False, needs_layout_passes=False
        ),
    )
    return sc_call(idx3, embp)


# speedup vs baseline: 1.6597x; 1.0572x over previous
import functools

import jax
import jax.numpy as jnp
from jax import lax
from jax.experimental import pallas as pl
from jax.experimental.pallas import tpu as pltpu
from jax.experimental.pallas import tpu_sc as plsc

_V = 1_000_000
_D = 64
_B = 4096
_S = 200
_LANES = 16
_TW = 16384

_info = plsc.get_sparse_core_info()
_NC, _NS = _info.num_cores, _info.num_subcores
_NW = _NC * _NS
_BPW = _B // _NW

_CHUNK_STARTS = tuple(range(0, _S - _LANES, _LANES)) + (_S - _LANES,)


def _widen_body(embT_ref, out_ref):
    x = embT_ref[...]
    out_ref[:, 0:_D] = x.T
    out_ref[:, _D:128] = jnp.zeros((_TW, _D), jnp.float32)


def _widen(embT):
    return pl.pallas_call(
        _widen_body,
        grid=(pl.cdiv(_V, _TW),),
        in_specs=[pl.BlockSpec((_D, _TW), lambda i: (0, i))],
        out_specs=pl.BlockSpec((_TW, 128), lambda i: (i, 0)),
        out_shape=jax.ShapeDtypeStruct((_V, 128), jnp.float32),
    )(embT)


def _body(idx_hbm, emb_hbm, out_hbm, tok_v, idxT_v, acc_v, sem):
    wid = lax.axis_index("s") * _NC + lax.axis_index("c")
    pltpu.sync_copy(idx_hbm.at[wid], tok_v)

    iota = lax.iota(jnp.int32, _LANES)
    jvecs = [(iota + c) * _BPW for c in _CHUNK_STARTS]

    def tr_row(b, carry):
        for c, jvec in zip(_CHUNK_STARTS, jvecs):
            x = tok_v[b, pl.ds(c, _LANES)] * 2
            plsc.store_scatter(idxT_v, [jvec + b], x)
        return carry

    lax.fori_loop(0, _BPW, tr_row, 0)

    zeros = jnp.zeros((_LANES,), jnp.float32)

    def zero_row(i, carry):
        for k in range(_D // _LANES):
            acc_v[i, pl.ds(k * _LANES, _LANES)] = zeros
        return carry

    lax.fori_loop(0, _BPW, zero_row, 0)

    def issue(j, carry):
        pltpu.async_copy(
            emb_hbm.at[idxT_v.at[pl.ds(j * _BPW, _BPW)]], acc_v, sem, add=True
        )
        return carry

    lax.fori_loop(0, _S, issue, 0)

    def drain(j, carry):
        pltpu.make_async_copy(
            emb_hbm.at[idxT_v.at[pl.ds(0, _BPW)]], acc_v, sem
        ).wait()
        return carry

    lax.fori_loop(0, _S, drain, 0)

    scale = jnp.full((_LANES,), 1.0 / _S, jnp.float32)

    def scale_row(i, carry):
        for k in range(_D // _LANES):
            sl = pl.ds(k * _LANES, _LANES)
            acc_v[i, sl] = acc_v[i, sl] * scale
        return carry

    lax.fori_loop(0, _BPW, scale_row, 0)

    pltpu.sync_copy(acc_v, out_hbm.at[pl.ds(wid * _BPW, _BPW)])


def kernel(text_tokens, emb):
    embp = _widen(emb.T).reshape(2 * _V, _D)
    idx3 = text_tokens.astype(jnp.int32).reshape(_NW, _BPW, _S)
    mesh = plsc.VectorSubcoreMesh(core_axis_name="c", subcore_axis_name="s")
    sc_call = pl.kernel(
        _body,
        out_type=jax.ShapeDtypeStruct((_B, _D), jnp.float32),
        mesh=mesh,
        scratch_types=[
            pltpu.VMEM((_BPW, _S), jnp.int32),
            pltpu.VMEM((_S * _BPW,), jnp.int32),
            pltpu.VMEM((_BPW, _D), jnp.float32),
            pltpu.SemaphoreType.DMA,
        ],
        compiler_params=pltpu.CompilerParams(
            use_tc_tiling_on_sc=False, needs_layout_passes=False
        ),
    )
    return sc_call(idx3, embp)
